# baseline (speedup 1.0000x reference)
import jax
import jax.numpy as jnp
from jax import lax
from jax.experimental import pallas as pl
from jax.experimental.pallas import tpu as pltpu

H = 384
W = 384
C = 192
HW = H * W
HWH = HW // 2
K = 30
NEG = float("-inf")
CONV_BN = 12288


def _conv_body(w_ref, b_ref, f_ref, o_ref):
    x = jnp.dot(w_ref[...], f_ref[...], preferred_element_type=jnp.float32)
    x = x + b_ref[0, 0]
    o_ref[...] = jnp.clip(jax.nn.sigmoid(x), 0.0001, 1.0 - 0.0001)


def _conv_center(w, b, f2):
    return pl.pallas_call(
        _conv_body,
        grid=(HWH // CONV_BN,),
        in_specs=[
            pl.BlockSpec((1, C), lambda i: (0, 0)),
            pl.BlockSpec(memory_space=pltpu.SMEM),
            pl.BlockSpec((C, CONV_BN), lambda i: (0, i)),
        ],
        out_specs=pl.BlockSpec((1, CONV_BN), lambda i: (0, i)),
        out_shape=jax.ShapeDtypeStruct((1, HWH), jnp.float32),
    )(w, b, f2)


def _shift_rows(x, dy, fill):
    if dy == 0:
        return x
    blk = jnp.full((abs(dy), x.shape[1]), fill, x.dtype)
    if dy > 0:
        return jnp.concatenate([x[dy:, :], blk], axis=0)
    return jnp.concatenate([blk, x[:dy, :]], axis=0)


def _shift_cols(x, dx, fill):
    if dx == 0:
        return x
    blk = jnp.full((x.shape[0], abs(dx)), fill, x.dtype)
    if dx > 0:
        return jnp.concatenate([x[:, dx:], blk], axis=1)
    return jnp.concatenate([blk, x[:, :dx]], axis=1)


def _select_body(c_ref, ft_ref, fb_ref, scores_ref, ys_ref, xs_ref, param_ref,
                 m_ref, rmax_ref, pos_ref, pgwin_ref, pg_ref, sem):
    c = c_ref[...]
    rowsum = c + _shift_cols(c, -1, 0.0) + _shift_cols(c, 1, 0.0)
    s = rowsum + _shift_rows(rowsum, -1, 0.0) + _shift_rows(rowsum, 1, 0.0)
    c2 = (c + s / 9.0) / 2.0
    rm = c2
    for dx in (-3, -2, -1, 1, 2, 3):
        rm = jnp.maximum(rm, _shift_cols(c2, dx, NEG))
    mm = rm
    for dy in (-3, -2, -1, 1, 2, 3):
        mm = jnp.maximum(mm, _shift_rows(rm, dy, NEG))
    masked = jnp.where(mm == c2, c2, 0.0)
    m_ref[...] = masked
    rmax_ref[...] = jnp.max(masked, axis=1, keepdims=True)

    lane_iota = lax.broadcasted_iota(jnp.int32, (1, W), 1)
    row_iota = lax.broadcasted_iota(jnp.int32, (H, 1), 0)

    def body(i, carry):
        rmax = rmax_ref[...]
        gmax = jnp.max(rmax)
        h = jnp.min(jnp.where(rmax == gmax, row_iota, H))
        row = m_ref[pl.ds(h, 1), :]
        wj = jnp.min(jnp.where(row == gmax, lane_iota, W))
        newrow = jnp.where(lane_iota == wj, NEG, row)
        m_ref[pl.ds(h, 1), :] = newrow
        rmax_ref[pl.ds(h, 1), :] = jnp.max(newrow, axis=1, keepdims=True)
        scores_ref[i] = gmax
        pos_ref[i] = h * W + wj
        ys_ref[i] = h
        xs_ref[i] = wj
        return carry

    lax.fori_loop(0, K, body, 0)

    for p in range(K):
        pos = pos_ref[p]
        istop = pos < HWH
        cpos = jnp.where(istop, pos, pos - HWH)
        base = pl.multiple_of((cpos >> 7) * 128, 128)
        dst = pgwin_ref.at[:, pl.ds(p * 128, 128)]

        @pl.when(istop)
        def _(base=base, dst=dst):
            pltpu.make_async_copy(ft_ref.at[:, pl.ds(base, 128)], dst, sem).start()

        @pl.when(jnp.logical_not(istop))
        def _(base=base, dst=dst):
            pltpu.make_async_copy(fb_ref.at[:, pl.ds(base, 128)], dst, sem).start()

    lane128 = lax.broadcasted_iota(jnp.int32, (1, 128), 1)
    for p in range(K):
        pltpu.make_async_copy(
            ft_ref.at[:, pl.ds(0, 128)],
            pgwin_ref.at[:, pl.ds(p * 128, 128)],
            sem,
        ).wait()
        win = pgwin_ref[:, pl.ds(p * 128, 128)]
        onehot = lane128 == (pos_ref[p] & 127)
        pg_ref[:, pl.ds(p, 1)] = jnp.sum(
            jnp.where(onehot, win, 0.0), axis=1, keepdims=True
        )
    eye = jnp.where(
        lax.broadcasted_iota(jnp.int32, (32, 32), 0)
        == lax.broadcasted_iota(jnp.int32, (32, 32), 1),
        1.0,
        0.0,
    )
    param_ref[...] = lax.dot_general(
        eye, pg_ref[...], (((1,), (1,)), ((), ())),
        precision=lax.Precision.HIGHEST,
        preferred_element_type=jnp.float32,
    )


def _select_topk(center, ft, fb):
    return pl.pallas_call(
        _select_body,
        in_specs=[
            pl.BlockSpec((H, W), lambda: (0, 0)),
            pl.BlockSpec(memory_space=pl.ANY),
            pl.BlockSpec(memory_space=pl.ANY),
        ],
        out_specs=[
            pl.BlockSpec(memory_space=pltpu.SMEM),
            pl.BlockSpec(memory_space=pltpu.SMEM),
            pl.BlockSpec(memory_space=pltpu.SMEM),
            pl.BlockSpec((32, C), lambda: (0, 0)),
        ],
        out_shape=[
            jax.ShapeDtypeStruct((32,), jnp.float32),
            jax.ShapeDtypeStruct((32,), jnp.int32),
            jax.ShapeDtypeStruct((32,), jnp.int32),
            jax.ShapeDtypeStruct((32, C), jnp.float32),
        ],
        scratch_shapes=[
            pltpu.VMEM((H, W), jnp.float32),
            pltpu.VMEM((H, 1), jnp.float32),
            pltpu.SMEM((32,), jnp.int32),
            pltpu.VMEM((C, K * 128), jnp.float32),
            pltpu.VMEM((C, 32), jnp.float32),
            pltpu.SemaphoreType.DMA,
        ],
    )(center, ft, fb)


def kernel(features, conv_w, conv_b):
    ft = features[0, :, : H // 2, :].reshape(C, HWH)
    fb = features[0, :, H // 2 :, :].reshape(C, HWH)
    w = conv_w[-1:]
    b = conv_b[-1:].reshape(1, 1)
    ctop = _conv_center(w, b, ft).reshape(H // 2, W)
    cbot = _conv_center(w, b, fb).reshape(H // 2, W)
    center = jnp.concatenate([ctop, cbot], axis=0)
    scores32, ys32, xs32, params32 = _select_topk(center, ft, fb)
    instance_coord = jnp.stack([ys32[:K], xs32[:K]], axis=1)
    instance_imgid = jnp.zeros((K,), jnp.int32)
    instance_param = params32[:K]
    scores = scores32[:K]
    return (instance_coord, instance_imgid, instance_param, scores)


# speedup vs baseline: 1.3802x; 1.3802x over previous
import jax
import jax.numpy as jnp
from jax import lax
from jax.experimental import pallas as pl
from jax.experimental.pallas import tpu as pltpu

H = 384
W = 384
C = 192
HW = H * W
HWH = HW // 2
K = 30
NEG = float("-inf")
CONV_BN = 24576


def _conv_body(w_ref, b_ref, f_ref, o_ref):
    x = jnp.dot(w_ref[...], f_ref[...], preferred_element_type=jnp.float32)
    x = x + b_ref[0, 0]
    o_ref[...] = jnp.clip(jax.nn.sigmoid(x), 0.0001, 1.0 - 0.0001)


def _conv_center(w, b, f2):
    return pl.pallas_call(
        _conv_body,
        grid=(HW // CONV_BN,),
        in_specs=[
            pl.BlockSpec((1, C), lambda i: (0, 0)),
            pl.BlockSpec(memory_space=pltpu.SMEM),
            pl.BlockSpec((C, CONV_BN), lambda i: (0, i)),
        ],
        out_specs=pl.BlockSpec((1, CONV_BN), lambda i: (0, i)),
        out_shape=jax.ShapeDtypeStruct((1, HW), jnp.float32),
    )(w, b, f2)


def _shift_rows(x, dy, fill):
    if dy == 0:
        return x
    blk = jnp.full((abs(dy), x.shape[1]), fill, x.dtype)
    if dy > 0:
        return jnp.concatenate([x[dy:, :], blk], axis=0)
    return jnp.concatenate([blk, x[:dy, :]], axis=0)


def _shift_cols(x, dx, fill):
    if dx == 0:
        return x
    blk = jnp.full((x.shape[0], abs(dx)), fill, x.dtype)
    if dx > 0:
        return jnp.concatenate([x[:, dx:], blk], axis=1)
    return jnp.concatenate([blk, x[:, :dx]], axis=1)


def _select_body(c_ref, f_ref, scores_ref, ys_ref, xs_ref, param_ref,
                 m_ref, rmax_ref, pos_ref, pgwin_ref, pg_ref, sem):
    c = c_ref[...]
    rowsum = c + _shift_cols(c, -1, 0.0) + _shift_cols(c, 1, 0.0)
    s = rowsum + _shift_rows(rowsum, -1, 0.0) + _shift_rows(rowsum, 1, 0.0)
    c2 = (c + s / 9.0) / 2.0
    rm = c2
    for dx in (-3, -2, -1, 1, 2, 3):
        rm = jnp.maximum(rm, _shift_cols(c2, dx, NEG))
    mm = rm
    for dy in (-3, -2, -1, 1, 2, 3):
        mm = jnp.maximum(mm, _shift_rows(rm, dy, NEG))
    masked = jnp.where(mm == c2, c2, 0.0)
    m_ref[...] = masked
    rmax_ref[...] = jnp.max(masked, axis=1, keepdims=True)

    lane_iota = lax.broadcasted_iota(jnp.int32, (1, W), 1)
    row_iota = lax.broadcasted_iota(jnp.int32, (H, 1), 0)

    def body(i, carry):
        rmax = rmax_ref[...]
        gmax = jnp.max(rmax)
        h = jnp.min(jnp.where(rmax == gmax, row_iota, H))
        row = m_ref[pl.ds(h, 1), :]
        wj = jnp.min(jnp.where(row == gmax, lane_iota, W))
        newrow = jnp.where(lane_iota == wj, NEG, row)
        m_ref[pl.ds(h, 1), :] = newrow
        rmax_ref[pl.ds(h, 1), :] = jnp.max(newrow, axis=1, keepdims=True)
        scores_ref[i] = gmax
        pos_ref[i] = h * W + wj
        ys_ref[i] = h
        xs_ref[i] = wj
        return carry

    lax.fori_loop(0, K, body, 0)

    copies = [
        pltpu.make_async_copy(
            f_ref.at[:, pl.ds(pl.multiple_of((pos_ref[p] >> 7) * 128, 128), 128)],
            pgwin_ref.at[:, pl.ds(p * 128, 128)],
            sem,
        )
        for p in range(K)
    ]
    for cp in copies:
        cp.start()
    lane128 = lax.broadcasted_iota(jnp.int32, (1, 128), 1)
    for p, cp in enumerate(copies):
        cp.wait()
        win = pgwin_ref[:, pl.ds(p * 128, 128)]
        onehot = lane128 == (pos_ref[p] & 127)
        pg_ref[:, pl.ds(p, 1)] = jnp.sum(
            jnp.where(onehot, win, 0.0), axis=1, keepdims=True
        )
    eye = jnp.where(
        lax.broadcasted_iota(jnp.int32, (32, 32), 0)
        == lax.broadcasted_iota(jnp.int32, (32, 32), 1),
        1.0,
        0.0,
    )
    param_ref[...] = lax.dot_general(
        eye, pg_ref[...], (((1,), (1,)), ((), ())),
        precision=lax.Precision.HIGHEST,
        preferred_element_type=jnp.float32,
    )


def _select_topk(center, f2):
    return pl.pallas_call(
        _select_body,
        in_specs=[
            pl.BlockSpec((H, W), lambda: (0, 0)),
            pl.BlockSpec(memory_space=pl.ANY),
        ],
        out_specs=[
            pl.BlockSpec(memory_space=pltpu.SMEM),
            pl.BlockSpec(memory_space=pltpu.SMEM),
            pl.BlockSpec(memory_space=pltpu.SMEM),
            pl.BlockSpec((32, C), lambda: (0, 0)),
        ],
        out_shape=[
            jax.ShapeDtypeStruct((32,), jnp.float32),
            jax.ShapeDtypeStruct((32,), jnp.int32),
            jax.ShapeDtypeStruct((32,), jnp.int32),
            jax.ShapeDtypeStruct((32, C), jnp.float32),
        ],
        scratch_shapes=[
            pltpu.VMEM((H, W), jnp.float32),
            pltpu.VMEM((H, 1), jnp.float32),
            pltpu.SMEM((32,), jnp.int32),
            pltpu.VMEM((C, K * 128), jnp.float32),
            pltpu.VMEM((C, 32), jnp.float32),
            pltpu.SemaphoreType.DMA,
        ],
    )(center, f2)


def kernel(features, conv_w, conv_b):
    f2 = features.reshape(C, HW)
    w = conv_w[-1:]
    b = conv_b[-1:].reshape(1, 1)
    center = _conv_center(w, b, f2).reshape(H, W)
    scores32, ys32, xs32, params32 = _select_topk(center, f2)
    instance_coord = jnp.stack([ys32[:K], xs32[:K]], axis=1)
    instance_imgid = jnp.zeros((K,), jnp.int32)
    instance_param = params32[:K]
    scores = scores32[:K]
    return (instance_coord, instance_imgid, instance_param, scores)


# speedup vs baseline: 1.3994x; 1.0139x over previous
import jax
import jax.numpy as jnp
from jax import lax
from jax.experimental import pallas as pl
from jax.experimental.pallas import tpu as pltpu

H = 384
W = 384
C = 192
HW = H * W
HWH = HW // 2
K = 30
NEG = float("-inf")
CONV_BN = 12288


def _conv_body(w_ref, b_ref, f_ref, o_ref):
    x = jnp.dot(w_ref[...], f_ref[...], preferred_element_type=jnp.float32)
    x = x + b_ref[0, 0]
    o_ref[...] = jnp.clip(jax.nn.sigmoid(x), 0.0001, 1.0 - 0.0001)


def _conv_center(w, b, f2):
    return pl.pallas_call(
        _conv_body,
        grid=(HW // CONV_BN,),
        in_specs=[
            pl.BlockSpec((1, C), lambda i: (0, 0)),
            pl.BlockSpec(memory_space=pltpu.SMEM),
            pl.BlockSpec((C, CONV_BN), lambda i: (0, i)),
        ],
        out_specs=pl.BlockSpec((1, CONV_BN), lambda i: (0, i)),
        out_shape=jax.ShapeDtypeStruct((1, HW), jnp.float32),
    )(w, b, f2)


def _shift_rows(x, dy, fill):
    if dy == 0:
        return x
    blk = jnp.full((abs(dy), x.shape[1]), fill, x.dtype)
    if dy > 0:
        return jnp.concatenate([x[dy:, :], blk], axis=0)
    return jnp.concatenate([blk, x[:dy, :]], axis=0)


def _shift_cols(x, dx, fill):
    if dx == 0:
        return x
    blk = jnp.full((x.shape[0], abs(dx)), fill, x.dtype)
    if dx > 0:
        return jnp.concatenate([x[:, dx:], blk], axis=1)
    return jnp.concatenate([blk, x[:, :dx]], axis=1)


def _select_body(c_ref, f_ref, scores_ref, ys_ref, xs_ref, param_ref,
                 m_ref, rmax_ref, pos_ref, pgwin_ref, pg_ref, sem):
    c = c_ref[...]
    rowsum = c + _shift_cols(c, -1, 0.0) + _shift_cols(c, 1, 0.0)
    s = rowsum + _shift_rows(rowsum, -1, 0.0) + _shift_rows(rowsum, 1, 0.0)
    c2 = (c + s / 9.0) / 2.0
    rm = c2
    for dx in (-3, -2, -1, 1, 2, 3):
        rm = jnp.maximum(rm, _shift_cols(c2, dx, NEG))
    mm = rm
    for dy in (-3, -2, -1, 1, 2, 3):
        mm = jnp.maximum(mm, _shift_rows(rm, dy, NEG))
    masked = jnp.where(mm == c2, c2, 0.0)
    m_ref[...] = masked
    rmax_ref[...] = jnp.max(masked, axis=1, keepdims=True)

    lane_iota = lax.broadcasted_iota(jnp.int32, (1, W), 1)
    row_iota = lax.broadcasted_iota(jnp.int32, (H, 1), 0)

    def body(i, carry):
        rmax = rmax_ref[...]
        gmax = jnp.max(rmax)
        h = jnp.min(jnp.where(rmax == gmax, row_iota, H))
        row = m_ref[pl.ds(h, 1), :]
        wj = jnp.min(jnp.where(row == gmax, lane_iota, W))
        newrow = jnp.where(lane_iota == wj, NEG, row)
        m_ref[pl.ds(h, 1), :] = newrow
        rmax_ref[pl.ds(h, 1), :] = jnp.max(newrow, axis=1, keepdims=True)
        scores_ref[i] = gmax
        pos = h * W + wj
        pos_ref[i] = pos
        ys_ref[i] = h
        xs_ref[i] = wj
        pltpu.make_async_copy(
            f_ref.at[:, pl.ds(pl.multiple_of((pos >> 7) * 128, 128), 128)],
            pgwin_ref.at[:, pl.ds(pl.multiple_of(i * 128, 128), 128)],
            sem,
        ).start()
        return carry

    lax.fori_loop(0, K, body, 0)

    lane128 = lax.broadcasted_iota(jnp.int32, (1, 128), 1)
    for p in range(K):
        pltpu.make_async_copy(
            f_ref.at[:, pl.ds(0, 128)],
            pgwin_ref.at[:, pl.ds(p * 128, 128)],
            sem,
        ).wait()
        win = pgwin_ref[:, pl.ds(p * 128, 128)]
        onehot = lane128 == (pos_ref[p] & 127)
        pg_ref[:, pl.ds(p, 1)] = jnp.sum(
            jnp.where(onehot, win, 0.0), axis=1, keepdims=True
        )
    eye = jnp.where(
        lax.broadcasted_iota(jnp.int32, (32, 32), 0)
        == lax.broadcasted_iota(jnp.int32, (32, 32), 1),
        1.0,
        0.0,
    )
    param_ref[...] = lax.dot_general(
        eye, pg_ref[...], (((1,), (1,)), ((), ())),
        precision=lax.Precision.HIGHEST,
        preferred_element_type=jnp.float32,
    )


def _select_topk(center, f2):
    return pl.pallas_call(
        _select_body,
        in_specs=[
            pl.BlockSpec((H, W), lambda: (0, 0)),
            pl.BlockSpec(memory_space=pl.ANY),
        ],
        out_specs=[
            pl.BlockSpec(memory_space=pltpu.SMEM),
            pl.BlockSpec(memory_space=pltpu.SMEM),
            pl.BlockSpec(memory_space=pltpu.SMEM),
            pl.BlockSpec((32, C), lambda: (0, 0)),
        ],
        out_shape=[
            jax.ShapeDtypeStruct((32,), jnp.float32),
            jax.ShapeDtypeStruct((32,), jnp.int32),
            jax.ShapeDtypeStruct((32,), jnp.int32),
            jax.ShapeDtypeStruct((32, C), jnp.float32),
        ],
        scratch_shapes=[
            pltpu.VMEM((H, W), jnp.float32),
            pltpu.VMEM((H, 1), jnp.float32),
            pltpu.SMEM((32,), jnp.int32),
            pltpu.VMEM((C, K * 128), jnp.float32),
            pltpu.VMEM((C, 32), jnp.float32),
            pltpu.SemaphoreType.DMA,
        ],
    )(center, f2)


def kernel(features, conv_w, conv_b):
    f2 = features.reshape(C, HW)
    w = conv_w[-1:]
    b = conv_b[-1:].reshape(1, 1)
    center = _conv_center(w, b, f2).reshape(H, W)
    scores32, ys32, xs32, params32 = _select_topk(center, f2)
    instance_coord = jnp.stack([ys32[:K], xs32[:K]], axis=1)
    instance_imgid = jnp.zeros((K,), jnp.int32)
    instance_param = params32[:K]
    scores = scores32[:K]
    return (instance_coord, instance_imgid, instance_param, scores)
